# Initial kernel scaffold; baseline (speedup 1.0000x reference)
import jax
import jax.numpy as jnp
from jax.experimental import pallas as pl


def kernel(x, batch, query, key_w, key_b, value_w, value_b):
    raise NotImplementedError("write your pallas kernel here")


# speedup vs baseline: 25.7601x; 25.7601x over previous
import functools

import jax
import jax.numpy as jnp
from jax import lax
from jax.experimental import pallas as pl
from jax.experimental.pallas import tpu as pltpu
from jax.experimental.pallas import tpu_sc as plsc

DIM = 128
NUM_HEADS = 4
HEAD_DIM = DIM // NUM_HEADS
SCALE = HEAD_DIM ** (-0.5)
EPAD = 16

ROWS_PER_BLOCK = 2000

NUM_WORKERS = 32
CHUNK_ROWS = 80


def _head_expand_matrix(dtype):
  row = lax.broadcasted_iota(jnp.int32, (EPAD, DIM), 0)
  col = lax.broadcasted_iota(jnp.int32, (EPAD, DIM), 1)
  return (col // HEAD_DIM == row).astype(dtype)


def _dense_body(x_ref, qkw_ref, qkb_ref, vw_ref, vb_ref, w_ref, e_ref):
  x = x_ref[...]
  logits = lax.dot_general(
      x, qkw_ref[...], (((1,), (1,)), ((), ())),
      precision=lax.Precision.HIGHEST,
      preferred_element_type=jnp.float32) + qkb_ref[...]
  e16 = jnp.exp(jnp.clip(logits, -20.0, 20.0))
  e_wide = lax.dot_general(
      e16, _head_expand_matrix(jnp.float32), (((1,), (0,)), ((), ())),
      precision=lax.Precision.HIGHEST,
      preferred_element_type=jnp.float32)
  v = lax.dot_general(
      x, vw_ref[...], (((1,), (1,)), ((), ())),
      precision=lax.Precision.HIGHEST,
      preferred_element_type=jnp.float32) + vb_ref[...]
  w_ref[...] = e_wide * v
  e_ref[...] = e16


def _sc_body(w_hbm, e_hbm, ids_hbm, zw_hbm, ze_hbm, outw_hbm, oute_hbm,
             ids_v, w_v, e_v, accw_s, acce_s, rows_per_worker, num_chunks):
  cid = lax.axis_index("c")
  sid = lax.axis_index("s")
  wid = sid * 2 + cid
  base = wid * rows_per_worker

  @pl.when(sid == 0)
  def _init():
    pltpu.sync_copy(zw_hbm, accw_s)
    pltpu.sync_copy(ze_hbm, acce_s)

  pltpu.sync_copy(ids_hbm.at[wid], ids_v)
  plsc.subcore_barrier()

  def chunk(j, carry):
    start = base + j * CHUNK_ROWS
    pltpu.sync_copy(w_hbm.at[pl.ds(start, CHUNK_ROWS)], w_v)
    pltpu.sync_copy(e_hbm.at[pl.ds(start, CHUNK_ROWS)], e_v)
    idx = ids_v.at[j]
    pltpu.sync_copy(w_v, accw_s.at[idx], add=True)
    pltpu.sync_copy(e_v, acce_s.at[idx], add=True)
    return carry

  lax.fori_loop(0, num_chunks, chunk, 0)
  plsc.subcore_barrier()

  @pl.when(sid == 0)
  def _flush():
    pltpu.sync_copy(accw_s, outw_hbm.at[cid])
    pltpu.sync_copy(acce_s, oute_hbm.at[cid])


def _norm_body(w0_ref, w1_ref, e0_ref, e1_ref, out_ref):
  w = w0_ref[...] + w1_ref[...]
  e = e0_ref[...] + e1_ref[...]
  s_wide = lax.dot_general(
      e, _head_expand_matrix(jnp.float32), (((1,), (0,)), ((), ())),
      precision=lax.Precision.HIGHEST,
      preferred_element_type=jnp.float32)
  out_ref[...] = w / (s_wide + 1e-8)


def kernel(x, batch, query, key_w, key_b, value_w, value_b):
  n, dim = x.shape
  assert dim == DIM
  b_seg = 1024

  batch = batch.astype(jnp.int32)

  q_full = (query.reshape(NUM_HEADS, 1, HEAD_DIM)
            * jnp.eye(NUM_HEADS, dtype=x.dtype).reshape(NUM_HEADS, NUM_HEADS, 1)
            ).reshape(NUM_HEADS, DIM)
  qk_w = SCALE * (q_full @ key_w)
  qk_b = SCALE * (q_full @ key_b)
  qk_w16 = jnp.zeros((EPAD, DIM), jnp.float32).at[:NUM_HEADS].set(qk_w)
  qk_b16 = jnp.zeros((1, EPAD), jnp.float32).at[0, :NUM_HEADS].set(qk_b)
  vb2 = value_b.reshape(1, DIM)

  num_blocks = n // ROWS_PER_BLOCK
  assert num_blocks * ROWS_PER_BLOCK == n

  w_arr, e_arr = pl.pallas_call(
      _dense_body,
      grid=(num_blocks,),
      in_specs=[
          pl.BlockSpec((ROWS_PER_BLOCK, DIM), lambda i: (i, 0)),
          pl.BlockSpec((EPAD, DIM), lambda i: (0, 0)),
          pl.BlockSpec((1, EPAD), lambda i: (0, 0)),
          pl.BlockSpec((DIM, DIM), lambda i: (0, 0)),
          pl.BlockSpec((1, DIM), lambda i: (0, 0)),
      ],
      out_specs=[
          pl.BlockSpec((ROWS_PER_BLOCK, DIM), lambda i: (i, 0)),
          pl.BlockSpec((ROWS_PER_BLOCK, EPAD), lambda i: (i, 0)),
      ],
      out_shape=[
          jax.ShapeDtypeStruct((n, DIM), jnp.float32),
          jax.ShapeDtypeStruct((n, EPAD), jnp.float32),
      ],
      compiler_params=pltpu.CompilerParams(
          dimension_semantics=("arbitrary",)),
  )(x, qk_w16, qk_b16, value_w, vb2)

  rows_per_worker = n // NUM_WORKERS
  assert rows_per_worker * NUM_WORKERS == n
  num_chunks = rows_per_worker // CHUNK_ROWS
  assert num_chunks * CHUNK_ROWS == rows_per_worker

  ids3d = batch.reshape(NUM_WORKERS, num_chunks, CHUNK_ROWS)
  zeros_w = jnp.zeros((b_seg, DIM), jnp.float32)
  zeros_e = jnp.zeros((b_seg, EPAD), jnp.float32)

  sc_kernel = pl.kernel(
      functools.partial(_sc_body, rows_per_worker=rows_per_worker,
                        num_chunks=num_chunks),
      out_type=(
          jax.ShapeDtypeStruct((2, b_seg, DIM), jnp.float32),
          jax.ShapeDtypeStruct((2, b_seg, EPAD), jnp.float32),
      ),
      mesh=plsc.VectorSubcoreMesh(core_axis_name="c", subcore_axis_name="s"),
      scratch_types=[
          pltpu.VMEM((num_chunks, CHUNK_ROWS), jnp.int32),
          pltpu.VMEM((CHUNK_ROWS, DIM), jnp.float32),
          pltpu.VMEM((CHUNK_ROWS, EPAD), jnp.float32),
          pltpu.VMEM_SHARED((b_seg, DIM), jnp.float32),
          pltpu.VMEM_SHARED((b_seg, EPAD), jnp.float32),
      ],
  )
  accw, acce = sc_kernel(w_arr, e_arr, ids3d, zeros_w, zeros_e)

  pooled = pl.pallas_call(
      _norm_body,
      out_shape=jax.ShapeDtypeStruct((b_seg, DIM), jnp.float32),
  )(accw[0], accw[1], acce[0], acce[1])
  return pooled


# speedup vs baseline: 50.6665x; 1.9669x over previous
import functools

import jax
import jax.numpy as jnp
from jax import lax
from jax.experimental import pallas as pl
from jax.experimental.pallas import tpu as pltpu
from jax.experimental.pallas import tpu_sc as plsc

DIM = 128
NUM_HEADS = 4
HEAD_DIM = DIM // NUM_HEADS
SCALE = HEAD_DIM ** (-0.5)
EPAD = 16

ROWS_PER_BLOCK = 2000

NUM_WORKERS = 32
CHUNK_ROWS = 80


def _head_expand(e16, rows):
  col = lax.broadcasted_iota(jnp.int32, (rows, DIM), 1)
  e_wide = jnp.where(col < HEAD_DIM, e16[:, 0:1], 0.0)
  for h in range(1, NUM_HEADS):
    in_head = (col >= h * HEAD_DIM) & (col < (h + 1) * HEAD_DIM)
    e_wide = e_wide + jnp.where(in_head, e16[:, h:h + 1], 0.0)
  return e_wide


def _dense_body(x_ref, wc_ref, qkb_ref, vb_ref, w_ref, e_ref):
  x = x_ref[...]
  y = lax.dot_general(
      x, wc_ref[...], (((1,), (0,)), ((), ())),
      precision=lax.Precision.DEFAULT,
      preferred_element_type=jnp.float32)
  v = y[:, :DIM] + vb_ref[...]
  logits = y[:, DIM:] + qkb_ref[...]
  e16 = jnp.exp(jnp.clip(logits, -20.0, 20.0))
  w_ref[...] = _head_expand(e16, x.shape[0]) * v
  e_ref[...] = e16


def _sc_body(w_hbm, e_hbm, ids_hbm, zw_hbm, ze_hbm, outw_hbm, oute_hbm,
             ids_v, w_v, e_v, accw_s, acce_s, gsem, *,
             rows_per_worker, num_chunks):
  cid = lax.axis_index("c")
  sid = lax.axis_index("s")
  wid = sid * 2 + cid
  base = wid * rows_per_worker

  @pl.when(sid == 0)
  def _init():
    pltpu.sync_copy(zw_hbm, accw_s)
    pltpu.sync_copy(ze_hbm, acce_s)

  def gather(j, buf):
    start = base + j * CHUNK_ROWS
    pltpu.async_copy(w_hbm.at[pl.ds(start, CHUNK_ROWS)], w_v.at[buf], gsem)
    pltpu.async_copy(e_hbm.at[pl.ds(start, CHUNK_ROWS)], e_v.at[buf], gsem)

  def wait_gather(buf):
    pltpu.make_async_copy(
        w_hbm.at[pl.ds(0, CHUNK_ROWS)], w_v.at[buf], gsem).wait()
    pltpu.make_async_copy(
        e_hbm.at[pl.ds(0, CHUNK_ROWS)], e_v.at[buf], gsem).wait()

  pltpu.sync_copy(ids_hbm.at[wid], ids_v)
  gather(0, 0)
  plsc.subcore_barrier()

  def chunk(j, carry):
    buf = lax.rem(j, 2)
    wait_gather(buf)
    @pl.when(j + 1 < num_chunks)
    def _prefetch():
      gather(j + 1, 1 - buf)

    idx = ids_v.at[j]
    pltpu.sync_copy(w_v.at[buf], accw_s.at[idx], add=True)
    pltpu.sync_copy(e_v.at[buf], acce_s.at[idx], add=True)
    return carry

  lax.fori_loop(0, num_chunks, chunk, 0)
  plsc.subcore_barrier()

  @pl.when(sid == 0)
  def _flush():
    pltpu.sync_copy(accw_s, outw_hbm.at[cid])
    pltpu.sync_copy(acce_s, oute_hbm.at[cid])


def _norm_body(w0_ref, w1_ref, e0_ref, e1_ref, out_ref):
  w = w0_ref[...] + w1_ref[...]
  e = e0_ref[...] + e1_ref[...]
  s_wide = _head_expand(e, w.shape[0])
  out_ref[...] = w / (s_wide + 1e-8)


def kernel(x, batch, query, key_w, key_b, value_w, value_b):
  n, dim = x.shape
  assert dim == DIM
  b_seg = 1024

  batch = batch.astype(jnp.int32)

  q_full = (query.reshape(NUM_HEADS, 1, HEAD_DIM)
            * jnp.eye(NUM_HEADS, dtype=x.dtype).reshape(NUM_HEADS, NUM_HEADS, 1)
            ).reshape(NUM_HEADS, DIM)
  qk_w = SCALE * (q_full @ key_w)
  qk_b = SCALE * (q_full @ key_b)
  qk_w16 = jnp.zeros((EPAD, DIM), jnp.float32).at[:NUM_HEADS].set(qk_w)
  qk_b16 = jnp.zeros((1, EPAD), jnp.float32).at[0, :NUM_HEADS].set(qk_b)
  vb2 = value_b.reshape(1, DIM)
  wc = jnp.concatenate([value_w.T, qk_w16.T], axis=1)

  num_blocks = n // ROWS_PER_BLOCK
  assert num_blocks * ROWS_PER_BLOCK == n

  w_arr, e_arr = pl.pallas_call(
      _dense_body,
      grid=(num_blocks,),
      in_specs=[
          pl.BlockSpec((ROWS_PER_BLOCK, DIM), lambda i: (i, 0)),
          pl.BlockSpec((DIM, DIM + EPAD), lambda i: (0, 0)),
          pl.BlockSpec((1, EPAD), lambda i: (0, 0)),
          pl.BlockSpec((1, DIM), lambda i: (0, 0)),
      ],
      out_specs=[
          pl.BlockSpec((ROWS_PER_BLOCK, DIM), lambda i: (i, 0)),
          pl.BlockSpec((ROWS_PER_BLOCK, EPAD), lambda i: (i, 0)),
      ],
      out_shape=[
          jax.ShapeDtypeStruct((n, DIM), jnp.float32),
          jax.ShapeDtypeStruct((n, EPAD), jnp.float32),
      ],
      compiler_params=pltpu.CompilerParams(
          dimension_semantics=("parallel",)),
  )(x, wc, qk_b16, vb2)

  rows_per_worker = n // NUM_WORKERS
  assert rows_per_worker * NUM_WORKERS == n
  num_chunks = rows_per_worker // CHUNK_ROWS
  assert num_chunks * CHUNK_ROWS == rows_per_worker

  ids3d = batch.reshape(NUM_WORKERS, num_chunks, CHUNK_ROWS)
  zeros_w = jnp.zeros((b_seg, DIM), jnp.float32)
  zeros_e = jnp.zeros((b_seg, EPAD), jnp.float32)

  sc_kernel = pl.kernel(
      functools.partial(_sc_body, rows_per_worker=rows_per_worker,
                        num_chunks=num_chunks),
      out_type=(
          jax.ShapeDtypeStruct((2, b_seg, DIM), jnp.float32),
          jax.ShapeDtypeStruct((2, b_seg, EPAD), jnp.float32),
      ),
      mesh=plsc.VectorSubcoreMesh(core_axis_name="c", subcore_axis_name="s"),
      scratch_types=[
          pltpu.VMEM((num_chunks, CHUNK_ROWS), jnp.int32),
          pltpu.VMEM((2, CHUNK_ROWS, DIM), jnp.float32),
          pltpu.VMEM((2, CHUNK_ROWS, EPAD), jnp.float32),
          pltpu.VMEM_SHARED((b_seg, DIM), jnp.float32),
          pltpu.VMEM_SHARED((b_seg, EPAD), jnp.float32),
          pltpu.SemaphoreType.DMA,
      ],
  )
  accw, acce = sc_kernel(w_arr, e_arr, ids3d, zeros_w, zeros_e)

  pooled = pl.pallas_call(
      _norm_body,
      out_shape=jax.ShapeDtypeStruct((b_seg, DIM), jnp.float32),
  )(accw[0], accw[1], acce[0], acce[1])
  return pooled


# speedup vs baseline: 59.0298x; 1.1651x over previous
import functools

import jax
import jax.numpy as jnp
from jax import lax
from jax.experimental import pallas as pl
from jax.experimental.pallas import tpu as pltpu
from jax.experimental.pallas import tpu_sc as plsc

DIM = 128
NUM_HEADS = 4
HEAD_DIM = DIM // NUM_HEADS
SCALE = HEAD_DIM ** (-0.5)
EPAD = 16

ROWS_PER_BLOCK = 8000

NUM_WORKERS = 32
CHUNK_ROWS = 80


def _head_expand(e16, rows):
  col = lax.broadcasted_iota(jnp.int32, (rows, DIM), 1)
  e_wide = jnp.where(col < HEAD_DIM, e16[:, 0:1], 0.0)
  for h in range(1, NUM_HEADS):
    in_head = (col >= h * HEAD_DIM) & (col < (h + 1) * HEAD_DIM)
    e_wide = e_wide + jnp.where(in_head, e16[:, h:h + 1], 0.0)
  return e_wide


def _dense_body(x_ref, wc_ref, qkb_ref, vb_ref, w_ref, e_ref):
  x = x_ref[...]
  y = lax.dot_general(
      x, wc_ref[...], (((1,), (0,)), ((), ())),
      precision=lax.Precision.DEFAULT,
      preferred_element_type=jnp.float32)
  v = y[:, :DIM] + vb_ref[...]
  logits = y[:, DIM:] + qkb_ref[...]
  e16 = jnp.exp(jnp.clip(logits, -20.0, 20.0))
  w_ref[...] = _head_expand(e16, x.shape[0]) * v
  e_ref[...] = e16


def _sc_body(w_hbm, e_hbm, ids_hbm, zw_hbm, ze_hbm, outw_hbm, oute_hbm,
             ids_v, w_v, e_v, accw_s, acce_s, gsem, ssem0, ssem1, *,
             rows_per_worker, num_chunks):
  cid = lax.axis_index("c")
  sid = lax.axis_index("s")
  wid = sid * 2 + cid
  base = wid * rows_per_worker
  del ssem1

  @pl.when(sid == 0)
  def _init():
    pltpu.sync_copy(zw_hbm, accw_s)
    pltpu.sync_copy(ze_hbm, acce_s)

  def gather(j, buf):
    start = base + j * CHUNK_ROWS
    pltpu.async_copy(w_hbm.at[pl.ds(start, CHUNK_ROWS)], w_v.at[buf], gsem)
    pltpu.async_copy(e_hbm.at[pl.ds(start, CHUNK_ROWS)], e_v.at[buf], gsem)

  def wait_gather(buf):
    pltpu.make_async_copy(
        w_hbm.at[pl.ds(0, CHUNK_ROWS)], w_v.at[buf], gsem).wait()
    pltpu.make_async_copy(
        e_hbm.at[pl.ds(0, CHUNK_ROWS)], e_v.at[buf], gsem).wait()

  def scatter(j, buf):
    idx = ids_v.at[j]
    pltpu.sync_copy(w_v.at[buf], accw_s.at[idx], add=True)
    pltpu.sync_copy(e_v.at[buf], acce_s.at[idx], add=True)

  pltpu.sync_copy(ids_hbm.at[wid], ids_v)
  gather(0, 0)
  plsc.subcore_barrier()

  def chunk(j, carry):
    buf = lax.rem(j, 2)
    wait_gather(buf)
    @pl.when(j + 1 < num_chunks)
    def _prefetch():
      gather(j + 1, 1 - buf)
    scatter(j, buf)
    return carry

  lax.fori_loop(0, num_chunks, chunk, 0)
  plsc.subcore_barrier()

  @pl.when(sid == 0)
  def _flush():
    pltpu.sync_copy(accw_s, outw_hbm.at[cid])
    pltpu.sync_copy(acce_s, oute_hbm.at[cid])


def _norm_body(w0_ref, w1_ref, e0_ref, e1_ref, out_ref):
  w = w0_ref[...] + w1_ref[...]
  e = e0_ref[...] + e1_ref[...]
  s_wide = _head_expand(e, w.shape[0])
  out_ref[...] = w / (s_wide + 1e-8)


def kernel(x, batch, query, key_w, key_b, value_w, value_b):
  n, dim = x.shape
  assert dim == DIM
  b_seg = 1024

  batch = batch.astype(jnp.int32)

  q_full = (query.reshape(NUM_HEADS, 1, HEAD_DIM)
            * jnp.eye(NUM_HEADS, dtype=x.dtype).reshape(NUM_HEADS, NUM_HEADS, 1)
            ).reshape(NUM_HEADS, DIM)
  qk_w = SCALE * (q_full @ key_w)
  qk_b = SCALE * (q_full @ key_b)
  qk_w16 = jnp.zeros((EPAD, DIM), jnp.float32).at[:NUM_HEADS].set(qk_w)
  qk_b16 = jnp.zeros((1, EPAD), jnp.float32).at[0, :NUM_HEADS].set(qk_b)
  vb2 = value_b.reshape(1, DIM)
  wc = jnp.concatenate([value_w.T, qk_w16.T], axis=1)

  num_blocks = n // ROWS_PER_BLOCK
  assert num_blocks * ROWS_PER_BLOCK == n

  w_arr, e_arr = pl.pallas_call(
      _dense_body,
      grid=(num_blocks,),
      in_specs=[
          pl.BlockSpec((ROWS_PER_BLOCK, DIM), lambda i: (i, 0)),
          pl.BlockSpec((DIM, DIM + EPAD), lambda i: (0, 0)),
          pl.BlockSpec((1, EPAD), lambda i: (0, 0)),
          pl.BlockSpec((1, DIM), lambda i: (0, 0)),
      ],
      out_specs=[
          pl.BlockSpec((ROWS_PER_BLOCK, DIM), lambda i: (i, 0)),
          pl.BlockSpec((ROWS_PER_BLOCK, EPAD), lambda i: (i, 0)),
      ],
      out_shape=[
          jax.ShapeDtypeStruct((n, DIM), jnp.float32),
          jax.ShapeDtypeStruct((n, EPAD), jnp.float32),
      ],
      compiler_params=pltpu.CompilerParams(
          dimension_semantics=("parallel",)),
  )(x, wc, qk_b16, vb2)

  rows_per_worker = n // NUM_WORKERS
  assert rows_per_worker * NUM_WORKERS == n
  num_chunks = rows_per_worker // CHUNK_ROWS
  assert num_chunks * CHUNK_ROWS == rows_per_worker

  ids3d = batch.reshape(NUM_WORKERS, num_chunks, CHUNK_ROWS)
  zeros_w = jnp.zeros((b_seg, DIM), jnp.float32)
  zeros_e = jnp.zeros((b_seg, EPAD), jnp.float32)

  sc_kernel = pl.kernel(
      functools.partial(_sc_body, rows_per_worker=rows_per_worker,
                        num_chunks=num_chunks),
      out_type=(
          jax.ShapeDtypeStruct((2, b_seg, DIM), jnp.float32),
          jax.ShapeDtypeStruct((2, b_seg, EPAD), jnp.float32),
      ),
      mesh=plsc.VectorSubcoreMesh(core_axis_name="c", subcore_axis_name="s"),
      scratch_types=[
          pltpu.VMEM((num_chunks, CHUNK_ROWS), jnp.int32),
          pltpu.VMEM((2, CHUNK_ROWS, DIM), jnp.float32),
          pltpu.VMEM((2, CHUNK_ROWS, EPAD), jnp.float32),
          pltpu.VMEM_SHARED((b_seg, DIM), jnp.float32),
          pltpu.VMEM_SHARED((b_seg, EPAD), jnp.float32),
          pltpu.SemaphoreType.DMA,
          pltpu.SemaphoreType.DMA,
          pltpu.SemaphoreType.DMA,
      ],
  )
  accw, acce = sc_kernel(w_arr, e_arr, ids3d, zeros_w, zeros_e)

  pooled = pl.pallas_call(
      _norm_body,
      out_shape=jax.ShapeDtypeStruct((b_seg, DIM), jnp.float32),
  )(accw[0], accw[1], acce[0], acce[1])
  return pooled
